# baseline (speedup 1.0000x reference)
import jax
import jax.numpy as jnp
from jax.experimental import pallas as pl


def _fused_mlp_kernel(x_ref, w1_ref, b1_ref, gamma_ref, beta_ref,
                      w2_ref, b2_ref, out_ref):
    x = x_ref[...]
    h = jax.lax.dot_general(
        x, w1_ref[...], (((1,), (1,)), ((), ())),
        preferred_element_type=jnp.float32,
    ) + b1_ref[...]

    n = h.shape[0]
    mean = jnp.sum(h, axis=0, keepdims=True) * (1.0 / n)
    centered = h - mean
    var = jnp.sum(centered * centered, axis=0, keepdims=True) * (1.0 / n)
    h = centered * jax.lax.rsqrt(var + 1e-5) * gamma_ref[...] + beta_ref[...]
    h = jnp.maximum(h, 0.0)

    logits = jax.lax.dot_general(
        h, w2_ref[...], (((1,), (1,)), ((), ())),
        preferred_element_type=jnp.float32,
    ) + b2_ref[...]

    m = jnp.max(logits, axis=1, keepdims=True)
    shifted = logits - m
    lse = jnp.log(jnp.sum(jnp.exp(shifted), axis=1, keepdims=True))
    out_ref[...] = shifted - lse


def kernel(x, edge_index, W1, b1, gamma, beta, W2, b2):
    del edge_index
    n = x.shape[0]
    nclass = W2.shape[0]
    return pl.pallas_call(
        _fused_mlp_kernel,
        out_shape=jax.ShapeDtypeStruct((n, nclass), jnp.float32),
    )(x, W1, b1.reshape(1, -1), gamma.reshape(1, -1), beta.reshape(1, -1),
      W2, b2.reshape(1, -1))
